# Initial kernel scaffold; baseline (speedup 1.0000x reference)
import jax
import jax.numpy as jnp
from jax.experimental import pallas as pl


def kernel(input, weight):
    raise NotImplementedError("write your pallas kernel here")


# speedup vs baseline: 4.8105x; 4.8105x over previous
import functools

import jax
import jax.numpy as jnp
from jax import lax
from jax.experimental import pallas as pl
from jax.experimental.pallas import tpu as pltpu
from jax.experimental.pallas import tpu_sc as plsc

EMB_DIM = 32
LANE = 128
CHUNK_BLKS = 8
NUM_WORKERS = 32


def _make_emb(nblk: int):
    per_worker = nblk // NUM_WORKERS
    chunks = per_worker // CHUNK_BLKS
    mesh = plsc.VectorSubcoreMesh(core_axis_name="c", subcore_axis_name="s")

    @functools.partial(
        pl.kernel,
        mesh=mesh,
        out_type=jax.ShapeDtypeStruct((nblk, LANE, EMB_DIM), jnp.float32),
        scratch_types=[
            pltpu.VMEM((CHUNK_BLKS, LANE), jnp.int32),
            pltpu.VMEM((CHUNK_BLKS, LANE, EMB_DIM), jnp.float32),
            pltpu.SemaphoreType.DMA,
        ],
        compiler_params=pltpu.CompilerParams(use_tc_tiling_on_sc=False),
    )
    def emb(idx_hbm, tab_hbm, out_hbm, idx_v, rows_v, sem):
        wid = lax.axis_index("s") * 2 + lax.axis_index("c")
        blk0 = wid * per_worker

        def chunk_body(g, carry):
            b0 = blk0 + g * CHUNK_BLKS
            pltpu.sync_copy(idx_hbm.at[pl.ds(b0, CHUNK_BLKS)], idx_v)
            copies = [
                pltpu.async_copy(tab_hbm.at[idx_v.at[j]], rows_v.at[j], sem)
                for j in range(CHUNK_BLKS)
            ]
            for c in copies:
                c.wait()
            pltpu.sync_copy(rows_v, out_hbm.at[pl.ds(b0, CHUNK_BLKS)])
            return carry

        lax.fori_loop(0, chunks, chunk_body, 0)

    return emb


def kernel(input, weight):
    batch, hist = input.shape
    total = batch * hist
    nblk = total // LANE
    idx = input.astype(jnp.int32).reshape(nblk, LANE)
    out = _make_emb(nblk)(idx, weight)
    return out.reshape(batch, hist, EMB_DIM)
